# Initial kernel scaffold; baseline (speedup 1.0000x reference)
import jax
import jax.numpy as jnp
from jax.experimental import pallas as pl


def kernel(qst, data, in_proj_w, in_proj_b, out_w, out_b, router_w, router_b, gp_w, gp_b):
    raise NotImplementedError("write your pallas kernel here")


# speedup vs baseline: 1.0393x; 1.0393x over previous
import functools

import jax
import jax.numpy as jnp
from jax import lax
from jax.experimental import pallas as pl
from jax.experimental.pallas import tpu as pltpu

D_MODEL = 2048
NHEAD = 16
N_EXPERTS = 16
TOPK_K = 8
SIGMA = 9.0
MARGIN_C = 1.0 / (N_EXPERTS * 2)
WCONST = 0.3989422804014327

_F32 = jnp.float32
_BF16 = jnp.bfloat16


def _q_body(qst_ref, wq_ref, bq_ref, q_exp_ref, *, nhead):
    qst = qst_ref[...]
    q = lax.dot_general(qst, wq_ref[...], (((1,), (1,)), ((), ())),
                        preferred_element_type=_F32) + bq_ref[0][None, :]
    b_sz, c_sz = q.shape
    dh = c_sz // nhead
    c_iota = lax.broadcasted_iota(jnp.int32, (b_sz, c_sz, nhead), 1)
    h_iota = lax.broadcasted_iota(jnp.int32, (b_sz, c_sz, nhead), 2)
    q_exp_ref[...] = jnp.where(c_iota // dh == h_iota, q[:, :, None], 0.0)


def _attn_body(data_ref, qexp_ref, wk_ref, bk_ref, wv_ref, bv_ref, s_ref,
               scores_ref, v_ref, *, nt, tt, nhead, scale):
    t = pl.program_id(1)
    d = data_ref[0]
    kt = lax.dot_general(d, wk_ref[...], (((1,), (1,)), ((), ())),
                         preferred_element_type=_F32) + bk_ref[0][None, :]
    vt = lax.dot_general(d, wv_ref[...], (((1,), (1,)), ((), ())),
                         preferred_element_type=_F32) + bv_ref[0][None, :]
    v_ref[pl.ds(t * tt, tt), :] = vt.astype(_BF16)
    sc = lax.dot_general(kt, qexp_ref[0], (((1,), (0,)), ((), ())),
                         preferred_element_type=_F32) * scale
    scores_ref[pl.ds(t * tt, tt), :] = sc

    @pl.when(t == nt - 1)
    def _():
        all_sc = scores_ref[...]
        m = jnp.max(all_sc, axis=0)
        p = jnp.exp(all_sc - m[None, :])
        attn = p / jnp.sum(p, axis=0)[None, :]
        attn_bf = attn.astype(_BF16)
        v_all = v_ref[...]
        ctx_full = lax.dot_general(attn_bf, v_all, (((0,), (0,)), ((), ())),
                                   preferred_element_type=_F32)
        c_sz = ctx_full.shape[1]
        dh = c_sz // nhead
        h_iota = lax.broadcasted_iota(jnp.int32, (nhead, c_sz), 0)
        c_iota = lax.broadcasted_iota(jnp.int32, (nhead, c_sz), 1)
        ctx = jnp.sum(jnp.where(c_iota // dh == h_iota, ctx_full, 0.0), axis=0)
        s_ref[0, 0] = ctx


def _head_body(ctx_ref, out_w_ref, out_b_ref, router_w_ref, router_b_ref,
               gpc_w_ref, gpc_b_ref, gpw_w_ref, gpw_b_ref, weight_ref,
               *, n_experts, topk, t_len):
    ctx = ctx_ref[...]
    temp = lax.dot_general(ctx, out_w_ref[...], (((1,), (1,)), ((), ())),
                           preferred_element_type=_F32) + out_b_ref[...][None, :]
    logits = lax.dot_general(temp, router_w_ref[...], (((1,), (1,)), ((), ())),
                             preferred_element_type=_F32) + router_b_ref[...][None, :]
    c_raw = lax.dot_general(temp, gpc_w_ref[...], (((1,), (1,)), ((), ())),
                            preferred_element_type=_F32) + gpc_b_ref[...][None, :]
    w_raw = lax.dot_general(temp, gpw_w_ref[...], (((1,), (1,)), ((), ())),
                            preferred_element_type=_F32) + gpw_b_ref[...][None, :]

    b_sz = logits.shape[0]
    iota_e = lax.broadcasted_iota(jnp.int32, (b_sz, n_experts), 1)
    base = MARGIN_C + iota_e.astype(_F32) * ((1.0 - 2.0 * MARGIN_C) / (n_experts - 1))
    centers_all = base + jnp.tanh(c_raw) * MARGIN_C
    widths_all = jax.nn.sigmoid(w_raw)

    grid_t = lax.broadcasted_iota(jnp.int32, (b_sz, t_len), 1).astype(_F32)
    grid_t = grid_t * (1.0 / (t_len - 1))

    work = logits
    for k in range(topk):
        m = jnp.max(work, axis=1, keepdims=True)
        hit = work == m
        idx = jnp.min(jnp.where(hit, iota_e, n_experts), axis=1)
        sel = iota_e == idx[:, None]
        c_k = jnp.sum(jnp.where(sel, centers_all, 0.0), axis=1)
        w_k = jnp.sum(jnp.where(sel, widths_all, 0.0), axis=1)
        work = jnp.where(sel, -jnp.inf, work)

        c_k = jnp.clip(c_k, 0.0, 1.0)
        w_k = jnp.maximum(w_k, 0.09) / SIGMA
        expo = -((grid_t - c_k[:, None]) ** 2) / (2.0 * w_k[:, None] ** 2)
        row = (WCONST / w_k)[:, None] * jnp.exp(expo)
        row = row / jnp.max(row, axis=1, keepdims=True)
        weight_ref[k] = row


def kernel(qst, data, in_proj_w, in_proj_b, out_w, out_b,
           router_w, router_b, gp_w, gp_b):
    B, T, C = data.shape
    H = NHEAD
    dh = C // H
    E = N_EXPERTS
    scale = 1.0 / (dh ** 0.5)

    wq = in_proj_w[:C]
    wk = in_proj_w[C:2 * C]
    wv = in_proj_w[2 * C:]
    bq = in_proj_b[:C].reshape(1, C)
    bk = in_proj_b[C:2 * C].reshape(1, C)
    bv = in_proj_b[2 * C:].reshape(1, C)
    gp_w_r = gp_w.reshape(E, 2, C)
    gpc_w = gp_w_r[:, 0, :]
    gpw_w = gp_w_r[:, 1, :]
    gp_b_r = gp_b.reshape(E, 2)
    gpc_b = gp_b_r[:, 0]
    gpw_b = gp_b_r[:, 1]

    q_exp = pl.pallas_call(
        functools.partial(_q_body, nhead=H),
        in_specs=[
            pl.BlockSpec((B, C), lambda: (0, 0)),
            pl.BlockSpec((C, C), lambda: (0, 0)),
            pl.BlockSpec((1, C), lambda: (0, 0)),
        ],
        out_specs=pl.BlockSpec((B, C, H), lambda: (0, 0, 0)),
        out_shape=jax.ShapeDtypeStruct((B, C, H), _F32),
    )(qst, wq, bq)

    TT = 512
    NT = T // TT
    s = pl.pallas_call(
        functools.partial(_attn_body, nt=NT, tt=TT, nhead=H, scale=scale),
        grid=(B, NT),
        in_specs=[
            pl.BlockSpec((1, TT, C), lambda b, t: (b, t, 0)),
            pl.BlockSpec((1, C, H), lambda b, t: (b, 0, 0)),
            pl.BlockSpec((C, C), lambda b, t: (0, 0)),
            pl.BlockSpec((1, C), lambda b, t: (0, 0)),
            pl.BlockSpec((C, C), lambda b, t: (0, 0)),
            pl.BlockSpec((1, C), lambda b, t: (0, 0)),
        ],
        out_specs=pl.BlockSpec((1, 1, C), lambda b, t: (b, 0, 0)),
        out_shape=jax.ShapeDtypeStruct((B, 1, C), _F32),
        scratch_shapes=[
            pltpu.VMEM((T, H), _F32),
            pltpu.VMEM((T, C), _BF16),
        ],
    )(data, q_exp, wk, bk, wv, bv)

    ctx = s.reshape(B, C)
    weight = pl.pallas_call(
        functools.partial(_head_body, n_experts=E, topk=TOPK_K, t_len=T),
        in_specs=[pl.BlockSpec(x.shape, lambda *_, _n=x.ndim: (0,) * _n)
                  for x in (ctx, out_w, out_b, router_w,
                            router_b, gpc_w, gpc_b, gpw_w, gpw_b)],
        out_specs=pl.BlockSpec((TOPK_K, B, T), lambda: (0, 0, 0)),
        out_shape=jax.ShapeDtypeStruct((TOPK_K, B, T), _F32),
    )(ctx, out_w, out_b, router_w, router_b,
      gpc_w, gpc_b, gpw_w, gpw_b)
    return weight.transpose(1, 0, 2)
